# baseline (speedup 1.0000x reference)
import functools

import jax
import jax.numpy as jnp
from jax import lax
from jax.experimental import pallas as pl
from jax.experimental.pallas import tpu as pltpu
from jax.experimental.pallas import tpu_sc as plsc

NN = 10000
NE = 320000
D = 128
NC = 2
NS = 16
NW = NC * NS
EW = NE // NW
K = 64
NWIN = 160
EWP = NWIN * K
PADW = EWP - EW
PADROWS = 128
NNP = NN + PADROWS
BW = 8
NBLK = NWIN // BW
NBUF = 4
RPT = 632

_mesh = plsc.VectorSubcoreMesh(
    core_axis_name="c", subcore_axis_name="s", num_cores=NC, num_subcores=NS)


@functools.partial(
    pl.kernel,
    out_type=(
        jax.ShapeDtypeStruct((NC, NN, D), jnp.float32),
        jax.ShapeDtypeStruct((NC, NNP), jnp.float32),
    ),
    mesh=_mesh,
    scratch_types=[
        pltpu.VMEM((BW, K), jnp.int32),
        pltpu.VMEM((BW, K), jnp.int32),
        pltpu.VMEM((BW, K), jnp.int32),
        pltpu.VMEM((BW, K), jnp.int32),
        pltpu.VMEM((K, D), jnp.float32),
        pltpu.VMEM((K, D), jnp.float32),
        pltpu.VMEM((K, D), jnp.float32),
        pltpu.VMEM((K, D), jnp.float32),
        pltpu.VMEM((BW * K,), jnp.float32),
        pltpu.VMEM((BW * K,), jnp.int32),
        pltpu.VMEM((BW * K,), jnp.int32),
        pltpu.VMEM((2048,), jnp.float32),
        pltpu.VMEM_SHARED((NNP, D), jnp.float32),
        pltpu.VMEM_SHARED((NNP,), jnp.float32),
        pltpu.SemaphoreType.DMA,
        pltpu.SemaphoreType.DMA,
        pltpu.SemaphoreType.DMA,
        pltpu.SemaphoreType.DMA,
        pltpu.SemaphoreType.DMA,
        pltpu.SemaphoreType.DMA,
        pltpu.SemaphoreType.DMA,
        pltpu.SemaphoreType.DMA,
        pltpu.SemaphoreType.DMA,
        pltpu.SemaphoreType.DMA,
        pltpu.SemaphoreType.DMA,
    ],
)
def _sc_aggregate(wh_hbm, src_rs, dst_rs, dst_rf, acc_out, cnt_out,
                  st0, st1, dt0, dt1, rows0, rows1, rows2, rows3, ones_v,
                  dc0, dc1, zc_v, acc_s, cnt_s, isem, gsem0, gsem1, gsem2, gsem3,
                  ssem0, ssem1, ssem2, ssem3, csem, zsem):
    src_t = [st0, st1]
    dst_t = [dt0, dt1]
    dst_c = [dc0, dc1]
    rows = [rows0, rows1, rows2, rows3]
    gsem = [gsem0, gsem1, gsem2, gsem3]
    ssem = [ssem0, ssem1, ssem2, ssem3]

    c = lax.axis_index("c")
    s = lax.axis_index("s")
    wid = c * NS + s

    def idx_fetch(b, tb):
        pltpu.async_copy(src_rs.at[wid, pl.ds(b * BW, BW)], src_t[tb], isem)
        pltpu.async_copy(dst_rs.at[wid, pl.ds(b * BW, BW)], dst_t[tb], isem)
        pltpu.async_copy(dst_rf.at[wid, pl.ds(b * BW * K, BW * K)],
                         dst_c[tb], isem)

    def idx_wait(tb):
        pltpu.make_async_copy(src_rs.at[0, pl.ds(0, BW)], src_t[tb],
                              isem).wait()
        pltpu.make_async_copy(dst_rs.at[0, pl.ds(0, BW)], dst_t[tb],
                              isem).wait()
        pltpu.make_async_copy(dst_rf.at[0, pl.ds(0, BW * K)], dst_c[tb],
                              isem).wait()

    idx_fetch(0, 0)

    zero16 = jnp.zeros((16,), jnp.float32)
    one16 = jnp.ones((16,), jnp.float32)
    for j in range(BW * K // 16):
        ones_v[pl.ds(j * 16, 16)] = one16
    for j in range(2048 // 16):
        zc_v[pl.ds(j * 16, 16)] = zero16

    zbuf = rows[0]

    def _zero_row(r, carry):
        for l in range(D // 16):
            zbuf[r, pl.ds(l * 16, 16)] = zero16
        return carry
    lax.fori_loop(0, K, _zero_row, 0)

    row0 = s * RPT
    nfull = RPT // K
    zcopies = []
    for j in range(nfull):
        zcopies.append((zbuf, acc_s.at[pl.ds(row0 + j * K, K)]))
    rem = RPT - nfull * K
    zcopies.append((zbuf.at[pl.ds(0, rem)],
                    acc_s.at[pl.ds(row0 + nfull * K, rem)]))
    for src, dst in zcopies:
        pltpu.async_copy(src, dst, zsem)

    zc0 = [(zbuf.at[pl.ds(0, NNP - NS * RPT)],
            acc_s.at[pl.ds(NS * RPT, NNP - NS * RPT)])]
    for j in range(4):
        zc0.append((zc_v, cnt_s.at[pl.ds(j * 2048, 2048)]))
    zc0.append((zc_v.at[pl.ds(0, NNP - 4 * 2048)],
                cnt_s.at[pl.ds(4 * 2048, NNP - 4 * 2048)]))

    @pl.when(s == 0)
    def _zero_tail():
        for src, dst in zc0:
            pltpu.async_copy(src, dst, zsem)

    for src, dst in zcopies:
        pltpu.make_async_copy(src, dst, zsem).wait()

    @pl.when(s == 0)
    def _zero_tail_wait():
        for src, dst in zc0:
            pltpu.make_async_copy(src, dst, zsem).wait()

    idx_wait(0)
    plsc.subcore_barrier()

    def gather(tb, j, p):
        pltpu.async_copy(wh_hbm.at[src_t[tb].at[j]], rows[p], gsem[p])

    for b0 in range(NBUF):
        gather(0, b0, b0)

    @pl.loop(0, NBLK)
    def _block(b):
        tb_tr = b % 2
        for tb in range(2):
            @pl.when(tb_tr == tb)
            def _do_block():
                @pl.when(b < NBLK - 1)
                def _prefetch():
                    idx_fetch(b + 1, 1 - tb)

                for j in range(BW):
                    p = j % NBUF
                    pltpu.make_async_copy(
                        wh_hbm.at[src_t[tb].at[j]], rows[p], gsem[p]).wait()
                    pltpu.async_copy(rows[p], acc_s.at[dst_t[tb].at[j]],
                                     ssem[p], add=True)
                    if j == 0:
                        @pl.when(b > 0)
                        def _cnt_wait_prev():
                            pltpu.make_async_copy(
                                ones_v, cnt_s.at[dst_c[tb]], csem).wait()

                        pltpu.async_copy(ones_v, cnt_s.at[dst_c[tb]],
                                         csem, add=True)
                    pltpu.make_async_copy(rows[p], acc_s.at[dst_t[tb].at[j]],
                                          ssem[p]).wait()
                    if j < BW - NBUF:
                        gather(tb, j + NBUF, p)
                    else:
                        nj = j + NBUF - BW

                        @pl.when(b < NBLK - 1)
                        def _issue_next():
                            if nj == 0:
                                idx_wait(1 - tb)
                            gather(1 - tb, nj, p)

    pltpu.make_async_copy(ones_v, cnt_s.at[dst_c[0]], csem).wait()
    plsc.subcore_barrier()

    @pl.when(s < NS - 1)
    def _acc_out_body():
        pltpu.sync_copy(acc_s.at[pl.ds(row0, RPT)],
                        acc_out.at[c, pl.ds(row0, RPT)])

    @pl.when(s == NS - 1)
    def _acc_out_last():
        pltpu.sync_copy(acc_s.at[pl.ds((NS - 1) * RPT, NN - (NS - 1) * RPT)],
                        acc_out.at[c, pl.ds((NS - 1) * RPT,
                                            NN - (NS - 1) * RPT)])

    @pl.when(s == 0)
    def _cnt_out():
        pltpu.sync_copy(cnt_s, cnt_out.at[c])


def _tc_linear(x, w, b):
    n = x.shape[0]
    blk = 2000

    def body(x_ref, w_ref, b_ref, o_ref):
        o_ref[...] = (
            jnp.dot(x_ref[...], w_ref[...], preferred_element_type=jnp.float32)
            + b_ref[...])

    return pl.pallas_call(
        body,
        grid=(n // blk,),
        in_specs=[
            pl.BlockSpec((blk, D), lambda i: (i, 0)),
            pl.BlockSpec((D, D), lambda i: (0, 0)),
            pl.BlockSpec((1, D), lambda i: (0, 0)),
        ],
        out_specs=pl.BlockSpec((blk, D), lambda i: (i, 0)),
        out_shape=jax.ShapeDtypeStruct((n, D), jnp.float32),
    )(x, w, b.reshape(1, D))


def _tc_mean_lrelu_linear(acc_p, cnt_t, w, b):
    blk = 2000

    def body(a_ref, c_ref, w_ref, b_ref, o_ref):
        ssum = a_ref[0] + a_ref[1]
        cnt = c_ref[...].sum(axis=1, keepdims=True)
        h = ssum / jnp.maximum(cnt, 1.0)
        h = jnp.where(h > 0, h, 0.01 * h)
        o_ref[...] = (
            jnp.dot(h, w_ref[...], preferred_element_type=jnp.float32)
            + b_ref[...])

    return pl.pallas_call(
        body,
        grid=(NN // blk,),
        in_specs=[
            pl.BlockSpec((NC, blk, D), lambda i: (0, i, 0)),
            pl.BlockSpec((blk, NC), lambda i: (i, 0)),
            pl.BlockSpec((D, D), lambda i: (0, 0)),
            pl.BlockSpec((1, D), lambda i: (0, 0)),
        ],
        out_specs=pl.BlockSpec((blk, D), lambda i: (i, 0)),
        out_shape=jax.ShapeDtypeStruct((NN, D), jnp.float32),
    )(acc_p, cnt_t, w, b.reshape(1, D))


def _tc_mean(acc_p, cnt_t):
    blk = 2000

    def body(a_ref, c_ref, o_ref):
        ssum = a_ref[0] + a_ref[1]
        cnt = c_ref[...].sum(axis=1, keepdims=True)
        o_ref[...] = ssum / jnp.maximum(cnt, 1.0)

    return pl.pallas_call(
        body,
        grid=(NN // blk,),
        in_specs=[
            pl.BlockSpec((NC, blk, D), lambda i: (0, i, 0)),
            pl.BlockSpec((blk, NC), lambda i: (i, 0)),
        ],
        out_specs=pl.BlockSpec((blk, D), lambda i: (i, 0)),
        out_shape=jax.ShapeDtypeStruct((NN, D), jnp.float32),
    )(acc_p, cnt_t)


def kernel(embed, edge_index, W1, b1, W2, b2):
    src = edge_index[0].reshape(NW, EW)
    dst = edge_index[1].reshape(NW, EW)
    pad_ids = jnp.arange(NW * PADW, dtype=jnp.int32).reshape(NW, PADW)
    pad_src = (pad_ids * 97) % NN
    pad_dst = NN + pad_ids % PADROWS
    src_rs = jnp.concatenate([src, pad_src], axis=1).reshape(NW, NWIN, K)
    dst_rs = jnp.concatenate([dst, pad_dst], axis=1).reshape(NW, NWIN, K)
    wh1 = _tc_linear(embed, W1, b1)
    dst_rf = dst_rs.reshape(NW, NWIN * K)
    acc1, cnt1 = _sc_aggregate(wh1, src_rs, dst_rs, dst_rf)
    cnt1_t = cnt1[:, :NN].T
    wh2 = _tc_mean_lrelu_linear(acc1, cnt1_t, W2, b2)
    acc2, _ = _sc_aggregate(wh2, src_rs, dst_rs, dst_rf)
    return _tc_mean(acc2, cnt1_t)


# speedup vs baseline: 1.0063x; 1.0063x over previous
import functools

import jax
import jax.numpy as jnp
from jax import lax
from jax.experimental import pallas as pl
from jax.experimental.pallas import tpu as pltpu
from jax.experimental.pallas import tpu_sc as plsc

NN = 10000
NE = 320000
D = 128
NC = 2
NS = 16
NW = NC * NS
EW = NE // NW
K = 64
NWIN = 160
EWP = NWIN * K
PADW = EWP - EW
PADROWS = 128
NNP = NN + PADROWS
BW = 8
NBLK = NWIN // BW
NBUF = 4
RPT = 632

_mesh = plsc.VectorSubcoreMesh(
    core_axis_name="c", subcore_axis_name="s", num_cores=NC, num_subcores=NS)


@functools.partial(
    pl.kernel,
    out_type=(
        jax.ShapeDtypeStruct((NC, NN, D), jnp.float32),
        jax.ShapeDtypeStruct((NC, NNP), jnp.float32),
    ),
    mesh=_mesh,
    scratch_types=[
        pltpu.VMEM((BW, K), jnp.int32),
        pltpu.VMEM((BW, K), jnp.int32),
        pltpu.VMEM((BW, K), jnp.int32),
        pltpu.VMEM((BW, K), jnp.int32),
        pltpu.VMEM((K, D), jnp.float32),
        pltpu.VMEM((K, D), jnp.float32),
        pltpu.VMEM((K, D), jnp.float32),
        pltpu.VMEM((K, D), jnp.float32),
        pltpu.VMEM((BW * K,), jnp.float32),
        pltpu.VMEM((BW * K,), jnp.int32),
        pltpu.VMEM((BW * K,), jnp.int32),
        pltpu.VMEM((2048,), jnp.float32),
        pltpu.VMEM_SHARED((NNP, D), jnp.float32),
        pltpu.VMEM_SHARED((NNP,), jnp.float32),
        pltpu.SemaphoreType.DMA,
        pltpu.SemaphoreType.DMA,
        pltpu.SemaphoreType.DMA,
        pltpu.SemaphoreType.DMA,
        pltpu.SemaphoreType.DMA,
        pltpu.SemaphoreType.DMA,
        pltpu.SemaphoreType.DMA,
        pltpu.SemaphoreType.DMA,
        pltpu.SemaphoreType.DMA,
        pltpu.SemaphoreType.DMA,
        pltpu.SemaphoreType.DMA,
    ],
)
def _sc_aggregate(wh_hbm, src_rs, dst_rs, dst_rf, acc_out, cnt_out,
                  st0, st1, dt0, dt1, rows0, rows1, rows2, rows3, ones_v,
                  dc0, dc1, zc_v, acc_s, cnt_s, isem, gsem0, gsem1, gsem2, gsem3,
                  ssem0, ssem1, ssem2, ssem3, csem, zsem):
    src_t = [st0, st1]
    dst_t = [dt0, dt1]
    dst_c = [dc0, dc1]
    rows = [rows0, rows1, rows2, rows3]
    gsem = [gsem0, gsem1, gsem2, gsem3]
    ssem = [ssem0, ssem1, ssem2, ssem3]

    c = lax.axis_index("c")
    s = lax.axis_index("s")
    wid = c * NS + s

    def idx_fetch(b, tb):
        pltpu.async_copy(src_rs.at[wid, pl.ds(b * BW, BW)], src_t[tb], isem)
        pltpu.async_copy(dst_rs.at[wid, pl.ds(b * BW, BW)], dst_t[tb], isem)
        pltpu.async_copy(dst_rf.at[wid, pl.ds(b * BW * K, BW * K)],
                         dst_c[tb], isem)

    def idx_wait(tb):
        pltpu.make_async_copy(src_rs.at[0, pl.ds(0, BW)], src_t[tb],
                              isem).wait()
        pltpu.make_async_copy(dst_rs.at[0, pl.ds(0, BW)], dst_t[tb],
                              isem).wait()
        pltpu.make_async_copy(dst_rf.at[0, pl.ds(0, BW * K)], dst_c[tb],
                              isem).wait()

    idx_fetch(0, 0)

    zero16 = jnp.zeros((16,), jnp.float32)
    one16 = jnp.ones((16,), jnp.float32)
    for j in range(BW * K // 16):
        ones_v[pl.ds(j * 16, 16)] = one16
    for j in range(2048 // 16):
        zc_v[pl.ds(j * 16, 16)] = zero16

    zbuf = rows[0]

    def _zero_row(r, carry):
        for l in range(D // 16):
            zbuf[r, pl.ds(l * 16, 16)] = zero16
        return carry
    lax.fori_loop(0, K, _zero_row, 0)

    row0 = s * RPT
    nfull = RPT // K
    zcopies = []
    for j in range(nfull):
        zcopies.append((zbuf, acc_s.at[pl.ds(row0 + j * K, K)]))
    rem = RPT - nfull * K
    zcopies.append((zbuf.at[pl.ds(0, rem)],
                    acc_s.at[pl.ds(row0 + nfull * K, rem)]))
    for src, dst in zcopies:
        pltpu.async_copy(src, dst, zsem)

    zc0 = [(zbuf.at[pl.ds(0, NNP - NS * RPT)],
            acc_s.at[pl.ds(NS * RPT, NNP - NS * RPT)])]
    for j in range(4):
        zc0.append((zc_v, cnt_s.at[pl.ds(j * 2048, 2048)]))
    zc0.append((zc_v.at[pl.ds(0, NNP - 4 * 2048)],
                cnt_s.at[pl.ds(4 * 2048, NNP - 4 * 2048)]))

    @pl.when(s == 0)
    def _zero_tail():
        for src, dst in zc0:
            pltpu.async_copy(src, dst, zsem)

    for src, dst in zcopies:
        pltpu.make_async_copy(src, dst, zsem).wait()

    @pl.when(s == 0)
    def _zero_tail_wait():
        for src, dst in zc0:
            pltpu.make_async_copy(src, dst, zsem).wait()

    idx_wait(0)
    plsc.subcore_barrier()

    def gather(tb, j, p):
        pltpu.async_copy(wh_hbm.at[src_t[tb].at[j]], rows[p], gsem[p])

    for b0 in range(NBUF):
        gather(0, b0, b0)

    @pl.loop(0, NBLK)
    def _block(b):
        tb_tr = b % 2
        for tb in range(2):
            @pl.when(tb_tr == tb)
            def _do_block():
                @pl.when(b < NBLK - 1)
                def _prefetch():
                    idx_fetch(b + 1, 1 - tb)

                for j in range(BW):
                    p = j % NBUF
                    pltpu.make_async_copy(
                        wh_hbm.at[src_t[tb].at[j]], rows[p], gsem[p]).wait()
                    pltpu.async_copy(rows[p], acc_s.at[dst_t[tb].at[j]],
                                     ssem[p], add=True)
                    if j == 0:
                        @pl.when(b > 0)
                        def _cnt_wait_prev():
                            pltpu.make_async_copy(
                                ones_v, cnt_s.at[dst_c[tb]], csem).wait()

                        pltpu.async_copy(ones_v, cnt_s.at[dst_c[tb]],
                                         csem, add=True)
                    pltpu.make_async_copy(rows[p], acc_s.at[dst_t[tb].at[j]],
                                          ssem[p]).wait()
                    if j < BW - NBUF:
                        gather(tb, j + NBUF, p)
                    else:
                        nj = j + NBUF - BW

                        @pl.when(b < NBLK - 1)
                        def _issue_next():
                            if nj == 0:
                                idx_wait(1 - tb)
                            gather(1 - tb, nj, p)

    pltpu.make_async_copy(ones_v, cnt_s.at[dst_c[0]], csem).wait()
    plsc.subcore_barrier()

    @pl.when(s < NS - 1)
    def _acc_out_body():
        pltpu.sync_copy(acc_s.at[pl.ds(row0, RPT)],
                        acc_out.at[c, pl.ds(row0, RPT)])

    @pl.when(s == NS - 1)
    def _acc_out_last():
        pltpu.sync_copy(acc_s.at[pl.ds((NS - 1) * RPT, NN - (NS - 1) * RPT)],
                        acc_out.at[c, pl.ds((NS - 1) * RPT,
                                            NN - (NS - 1) * RPT)])

    @pl.when(s == 0)
    def _cnt_out():
        pltpu.sync_copy(cnt_s, cnt_out.at[c])


def _tc_linear(x, w, b):
    n = x.shape[0]
    blk = 10000

    def body(x_ref, w_ref, b_ref, o_ref):
        o_ref[...] = (
            jnp.dot(x_ref[...], w_ref[...], preferred_element_type=jnp.float32)
            + b_ref[...])

    return pl.pallas_call(
        body,
        grid=(n // blk,),
        in_specs=[
            pl.BlockSpec((blk, D), lambda i: (i, 0)),
            pl.BlockSpec((D, D), lambda i: (0, 0)),
            pl.BlockSpec((1, D), lambda i: (0, 0)),
        ],
        out_specs=pl.BlockSpec((blk, D), lambda i: (i, 0)),
        out_shape=jax.ShapeDtypeStruct((n, D), jnp.float32),
    )(x, w, b.reshape(1, D))


def _tc_mean_lrelu_linear(acc_p, cnt_t, w, b):
    blk = 10000

    def body(a_ref, c_ref, w_ref, b_ref, o_ref):
        ssum = a_ref[0] + a_ref[1]
        cnt = c_ref[...].sum(axis=1, keepdims=True)
        h = ssum / jnp.maximum(cnt, 1.0)
        h = jnp.where(h > 0, h, 0.01 * h)
        o_ref[...] = (
            jnp.dot(h, w_ref[...], preferred_element_type=jnp.float32)
            + b_ref[...])

    return pl.pallas_call(
        body,
        grid=(NN // blk,),
        in_specs=[
            pl.BlockSpec((NC, blk, D), lambda i: (0, i, 0)),
            pl.BlockSpec((blk, NC), lambda i: (i, 0)),
            pl.BlockSpec((D, D), lambda i: (0, 0)),
            pl.BlockSpec((1, D), lambda i: (0, 0)),
        ],
        out_specs=pl.BlockSpec((blk, D), lambda i: (i, 0)),
        out_shape=jax.ShapeDtypeStruct((NN, D), jnp.float32),
    )(acc_p, cnt_t, w, b.reshape(1, D))


def _tc_mean(acc_p, cnt_t):
    blk = 10000

    def body(a_ref, c_ref, o_ref):
        ssum = a_ref[0] + a_ref[1]
        cnt = c_ref[...].sum(axis=1, keepdims=True)
        o_ref[...] = ssum / jnp.maximum(cnt, 1.0)

    return pl.pallas_call(
        body,
        grid=(NN // blk,),
        in_specs=[
            pl.BlockSpec((NC, blk, D), lambda i: (0, i, 0)),
            pl.BlockSpec((blk, NC), lambda i: (i, 0)),
        ],
        out_specs=pl.BlockSpec((blk, D), lambda i: (i, 0)),
        out_shape=jax.ShapeDtypeStruct((NN, D), jnp.float32),
    )(acc_p, cnt_t)


def kernel(embed, edge_index, W1, b1, W2, b2):
    src = edge_index[0].reshape(NW, EW)
    dst = edge_index[1].reshape(NW, EW)
    pad_ids = jnp.arange(NW * PADW, dtype=jnp.int32).reshape(NW, PADW)
    pad_src = (pad_ids * 97) % NN
    pad_dst = NN + pad_ids % PADROWS
    src_rs = jnp.concatenate([src, pad_src], axis=1).reshape(NW, NWIN, K)
    dst_rs = jnp.concatenate([dst, pad_dst], axis=1).reshape(NW, NWIN, K)
    wh1 = _tc_linear(embed, W1, b1)
    dst_rf = dst_rs.reshape(NW, NWIN * K)
    acc1, cnt1 = _sc_aggregate(wh1, src_rs, dst_rs, dst_rf)
    cnt1_t = cnt1[:, :NN].T
    wh2 = _tc_mean_lrelu_linear(acc1, cnt1_t, W2, b2)
    acc2, _ = _sc_aggregate(wh2, src_rs, dst_rs, dst_rf)
    return _tc_mean(acc2, cnt1_t)
